# baseline (speedup 1.0000x reference)
import functools

import jax
import jax.numpy as jnp
from jax import lax
from jax.experimental import pallas as pl
from jax.experimental.pallas import tpu as pltpu
from jax.experimental.pallas import tpu_sc as plsc

VOCAB = 1000000
EMBED_DIM = 64
BATCH = 4096
SEQ = 200
SEQ_PAD = 208
HALF = SEQ_PAD // 2
NCHUNK = HALF // 16
DC = EMBED_DIM // 16

_info = plsc.get_sparse_core_info()
NC = _info.num_cores
NS = _info.num_subcores
NW = NC * NS
DPW = BATCH // NW

_mesh = plsc.VectorSubcoreMesh(core_axis_name="c", subcore_axis_name="s")


def _count_half(idx_ref, d, h):
    cnt = jnp.zeros((16,), jnp.float32)
    for k in range(NCHUNK):
        v = idx_ref[d, h, pl.ds(k * 16, 16)]
        cnt = cnt + jnp.where(v != 0, 1.0, 0.0).astype(jnp.float32)
    tail = idx_ref[d, h, pl.ds(HALF - 16, 16)]
    lane = lax.iota(jnp.int32, 16)
    tcnt = jnp.where((lane >= 8) & (tail != 0), 1.0, 0.0).astype(jnp.float32)
    return jnp.sum(cnt + tcnt)


def _accum_half(rows_ref, acc):
    def body(r, acc):
        return tuple(
            acc[c] + rows_ref[r, pl.ds(c * 16, 16)] for c in range(DC)
        )
    return lax.fori_loop(0, HALF, body, acc)


@functools.partial(
    pl.kernel,
    mesh=_mesh,
    out_type=jax.ShapeDtypeStruct((BATCH,), jnp.float32),
    scratch_types=[
        pltpu.VMEM((DPW, 2, HALF), jnp.int32),
        pltpu.VMEM((HALF, EMBED_DIM), jnp.float32),
        pltpu.VMEM((HALF, EMBED_DIM), jnp.float32),
        pltpu.VMEM((DPW,), jnp.float32),
        pltpu.VMEM((EMBED_DIM,), jnp.float32),
        pltpu.VMEM((EMBED_DIM,), jnp.float32),
        pltpu.VMEM((16,), jnp.float32),
        pltpu.SemaphoreType.DMA,
    ],
    compiler_params=pltpu.CompilerParams(needs_layout_passes=False,
                                         use_tc_tiling_on_sc=False),
)
def _emb_pool_kernel(docs_hbm, table_hbm, w_hbm, b_hbm, out_hbm,
                     idx_v, rows_a, rows_b, out_v, w_v, t0_v, b_v, sem):
    wid = lax.axis_index("s") * NC + lax.axis_index("c")
    base = wid * DPW

    pltpu.sync_copy(docs_hbm.at[pl.ds(base, DPW)], idx_v)
    pltpu.sync_copy(w_hbm, w_v)
    pltpu.sync_copy(table_hbm.at[0], t0_v)
    pltpu.sync_copy(b_hbm, b_v)

    w = [w_v[pl.ds(c * 16, 16)] for c in range(DC)]
    t0 = [t0_v[pl.ds(c * 16, 16)] for c in range(DC)]
    bvec = b_v[pl.ds(0, 16)]
    lane = lax.iota(jnp.int32, 16)

    def doc_body(d, carry):
        pltpu.async_copy(table_hbm.at[idx_v.at[d, 0]], rows_a, sem).wait()
        pltpu.async_copy(table_hbm.at[idx_v.at[d, 1]], rows_b, sem).wait()

        acc = tuple(jnp.zeros((16,), jnp.float32) for _ in range(DC))
        acc = _accum_half(rows_a, acc)
        acc = _accum_half(rows_b, acc)

        count = _count_half(idx_v, d, 0) + _count_half(idx_v, d, 1)
        countv = jnp.full((16,), count, jnp.float32)
        n0v = jnp.full((16,), jnp.float32(SEQ_PAD)) - countv
        invv = 1.0 / jnp.maximum(countv, jnp.full((16,), 1.0, jnp.float32))

        dot = jnp.zeros((16,), jnp.float32)
        for c in range(DC):
            pooled_c = (acc[c] - n0v * t0[c]) * invv
            dot = dot + pooled_c * w[c]
        lv = jnp.full((16,), jnp.sum(dot), jnp.float32) + bvec

        e = jnp.exp(lv - lv)
        val = e / e
        plsc.store_scatter(out_v, [jnp.full((16,), d, jnp.int32)], val,
                           mask=lane == 0)
        return carry

    lax.fori_loop(0, DPW, doc_body, 0)
    pltpu.sync_copy(out_v, out_hbm.at[pl.ds(base, DPW)])


def kernel(documents, table, W, b):
    docs = jnp.pad(documents.astype(jnp.int32), ((0, 0), (0, SEQ_PAD - SEQ)))
    docs = docs.reshape(BATCH, 2, HALF)
    out = _emb_pool_kernel(docs, table, W.reshape(EMBED_DIM),
                           jnp.full((16,), b[0], jnp.float32))
    return out.reshape(BATCH, 1)
